# baseline (speedup 1.0000x reference)
import functools

import jax
import jax.numpy as jnp
from jax import lax
from jax.experimental import pallas as pl
from jax.experimental.pallas import tpu as pltpu
from jax.experimental.pallas import tpu_sc as plsc

DIM = 768
E = 8
DFF = 1536
T = 2048
BLK = 128
NBLK = T // BLK + E
P = NBLK * BLK


def _cumsum_rows(x):
  n = x.shape[0]
  row = lax.broadcasted_iota(jnp.int32, x.shape, 0)
  s = 1
  while s < n:
    shifted = pltpu.roll(x, s, 0)
    x = x + jnp.where(row >= s, shifted, 0)
    s *= 2
  return x


def _cumsum_lanes(x):
  n = x.shape[1]
  col = lax.broadcasted_iota(jnp.int32, x.shape, 1)
  s = 1
  while s < n:
    shifted = pltpu.roll(x, s, 1)
    x = x + jnp.where(col >= s, shifted, 0)
    s *= 2
  return x


def _router_body(x_ref, rw_ref, dest_ref, w_ref, bmap_ref, aux_ref, lb_ref):
  xv = x_ref[...]
  rw = rw_ref[...]
  logits = lax.dot_general(xv, rw, (((1,), (1,)), ((), ())),
                           preferred_element_type=jnp.float32)
  m = jnp.max(logits, axis=1, keepdims=True)
  ex = jnp.exp(logits - m)
  probs = ex / jnp.sum(ex, axis=1, keepdims=True)
  pmax = jnp.max(probs, axis=1, keepdims=True)
  lane = lax.broadcasted_iota(jnp.int32, (T, E), 1)
  topi = jnp.min(jnp.where(probs == pmax, lane, E), axis=1, keepdims=True)
  w_ref[...] = jnp.broadcast_to(pmax / (pmax + 1e-6), (T, 128))

  onehot = (lane == topi).astype(jnp.int32)
  csum = _cumsum_rows(onehot)
  rank = jnp.sum((csum - onehot) * onehot, axis=1, keepdims=True)
  counts = jnp.sum(onehot, axis=0, keepdims=True)
  asz = ((counts + (BLK - 1)) // BLK) * BLK
  ends = _cumsum_lanes(asz)
  offs = ends - asz
  off_t = jnp.sum(offs * onehot, axis=1, keepdims=True)
  dest_ref[...] = off_t + rank

  bstart = lax.broadcasted_iota(jnp.int32, (NBLK, E), 0) * BLK
  bmap = jnp.sum((bstart >= jnp.broadcast_to(ends, (NBLK, E))).astype(jnp.int32),
                 axis=1, keepdims=True)
  bmap = jnp.minimum(bmap, E - 1)
  bmap_ref[...] = bmap

  rowb = lax.broadcasted_iota(jnp.int32, (NBLK, 1), 0)
  sw = ((bmap != pltpu.roll(bmap, 1, 0)) | (rowb == 0)).astype(jnp.int32)
  runidx = _cumsum_rows(sw) - 1
  buf = lax.rem(runidx, 3)
  lane_e = lax.broadcasted_iota(jnp.int32, (1, E), 1)
  has_pad = (ends[:, E - 1:E] < P).astype(jnp.int32)
  is_run_val = ((asz > 0) | ((lane_e == E - 1) & (has_pad > 0)))
  cand = jnp.where(is_run_val & (lane_e > bmap), lane_e, 99)
  nxt = jnp.min(cand, axis=1, keepdims=True)
  cand2 = jnp.where(is_run_val & (lane_e > nxt), lane_e, 99)
  nxt2 = jnp.min(cand2, axis=1, keepdims=True)
  do_pref = ((sw == 1) & (nxt2 < 99)).astype(jnp.int32)
  first_nxt = jnp.where(nxt < 99, nxt, 0)
  has_nxt = (nxt < 99).astype(jnp.int32)
  aux_ref[...] = jnp.concatenate(
      [sw, buf, jnp.minimum(nxt2, E - 1), do_pref, first_nxt, has_nxt],
      axis=1)

  psum = jnp.sum(probs, axis=0, keepdims=True)
  frac = counts.astype(jnp.float32) / (jnp.float32(T) + 1e-6)
  lb_ref[...] = jnp.sum(frac * psum, keepdims=True).reshape(1, 1) * E


def _router(xf, router_w):
  return pl.pallas_call(
      _router_body,
      out_shape=(
          jax.ShapeDtypeStruct((T, 1), jnp.int32),
          jax.ShapeDtypeStruct((T, 128), jnp.float32),
          jax.ShapeDtypeStruct((NBLK, 1), jnp.int32),
          jax.ShapeDtypeStruct((NBLK, 6), jnp.int32),
          jax.ShapeDtypeStruct((1, 1), jnp.float32),
      ),
  )(xf, router_w)


def _ffn_body(bmap_ref, aux_ref, xs_ref, f1w_hbm, f1b_ref, gw_hbm, gb_ref,
              f2w_hbm, f2b_ref, ws_ref, o_ref, w1_v, wg_v, w2_v, sems):
  i = pl.program_id(0)
  rs = aux_ref[i, 0]
  b = aux_ref[i, 1]
  ne2 = aux_ref[i, 2]
  dp = aux_ref[i, 3]
  cur_e = bmap_ref[i]

  def copy(hbm, vbuf, e, slot, k):
    return pltpu.make_async_copy(hbm.at[e], vbuf.at[slot], sems.at[slot, k])

  def copies(e, slot):
    return (
        copy(f1w_hbm, w1_v, e, slot, 0),
        copy(gw_hbm, wg_v, e, slot, 1),
        copy(f2w_hbm, w2_v, e, slot, 2),
    )

  @pl.when(i == 0)
  def _():
    for c in copies(cur_e, b):
      c.start()

    @pl.when(aux_ref[0, 5] == 1)
    def _():
      for c in copies(aux_ref[0, 4], 1):
        c.start()

  @pl.when((rs == 1) & (dp == 1))
  def _():
    for c in copies(ne2, lax.rem(b + 2, 3)):
      c.start()

  bf = jnp.bfloat16
  xb = xs_ref[...].astype(bf)
  nt = (((1,), (1,)), ((), ()))

  @pl.when(rs == 1)
  def _():
    copy(f1w_hbm, w1_v, cur_e, b, 0).wait()

  w1 = w1_v[pl.ds(b, 1)][0].astype(bf)
  h = lax.dot_general(xb, w1, nt, preferred_element_type=jnp.float32)
  h = h + f1b_ref[0]

  @pl.when(rs == 1)
  def _():
    copy(gw_hbm, wg_v, cur_e, b, 1).wait()

  wg = wg_v[pl.ds(b, 1)][0].astype(bf)
  g = lax.dot_general(xb, wg, nt, preferred_element_type=jnp.float32)
  g = g + gb_ref[0]
  a = g * jax.nn.sigmoid(g) * h

  @pl.when(rs == 1)
  def _():
    copy(f2w_hbm, w2_v, cur_e, b, 2).wait()

  w2 = w2_v[pl.ds(b, 1)][0].astype(bf)
  o = lax.dot_general(a.astype(bf), w2, nt, preferred_element_type=jnp.float32)
  o_ref[...] = (o + f2b_ref[0]) * ws_ref[...][:, 0:1]


def _ffn(bmap, aux, xs, fc1_w, fc1_b, gate_w, gate_b, fc2_w, fc2_b, ws8):
  grid_spec = pltpu.PrefetchScalarGridSpec(
      num_scalar_prefetch=2,
      grid=(NBLK,),
      in_specs=[
          pl.BlockSpec((BLK, DIM), lambda i, bm, ax: (i, 0)),
          pl.BlockSpec(memory_space=pl.MemorySpace.ANY),
          pl.BlockSpec((1, 1, DFF), lambda i, bm, ax: (bm[i], 0, 0)),
          pl.BlockSpec(memory_space=pl.MemorySpace.ANY),
          pl.BlockSpec((1, 1, DFF), lambda i, bm, ax: (bm[i], 0, 0)),
          pl.BlockSpec(memory_space=pl.MemorySpace.ANY),
          pl.BlockSpec((1, 1, DIM), lambda i, bm, ax: (bm[i], 0, 0)),
          pl.BlockSpec((BLK, 128), lambda i, bm, ax: (i, 0)),
      ],
      out_specs=pl.BlockSpec((BLK, DIM), lambda i, bm, ax: (i, 0)),
      scratch_shapes=[
          pltpu.VMEM((3, DFF, DIM), jnp.float32),
          pltpu.VMEM((3, DFF, DIM), jnp.float32),
          pltpu.VMEM((3, DIM, DFF), jnp.float32),
          pltpu.SemaphoreType.DMA((3, 3)),
      ],
  )
  return pl.pallas_call(
      _ffn_body,
      grid_spec=grid_spec,
      out_shape=jax.ShapeDtypeStruct((P, DIM), jnp.float32),
  )(bmap, aux, xs, fc1_w, fc1_b, gate_w, gate_b, fc2_w, fc2_b, ws8)


@functools.cache
def _sc_geometry():
  info = plsc.get_sparse_core_info()
  nc, ns = info.num_cores, info.num_subcores
  return nc, ns, T // (nc * ns)


def _dispatch(xf, dest, wrow):
  _NC, _NS, _CHUNK = _sc_geometry()
  mesh = plsc.VectorSubcoreMesh(core_axis_name="c", subcore_axis_name="s")

  @functools.partial(
      pl.kernel,
      mesh=mesh,
      out_type=(
          jax.ShapeDtypeStruct((P, DIM), jnp.float32),
          jax.ShapeDtypeStruct((P, 128), jnp.float32),
      ),
      scratch_types=[
          pltpu.VMEM((_CHUNK,), jnp.int32),
          pltpu.VMEM((_CHUNK, DIM), jnp.float32),
          pltpu.VMEM((_CHUNK, 128), jnp.float32),
          pltpu.SemaphoreType.DMA,
          pltpu.SemaphoreType.DMA,
      ],
  )
  def k(x_hbm, dest_hbm, w_hbm, xs_hbm, ws8_hbm, idx_v, rows_v, wrows_v, sem,
        sem2):
    wid = lax.axis_index("s") * _NC + lax.axis_index("c")
    base = wid * _CHUNK
    pltpu.sync_copy(dest_hbm.at[pl.ds(base, _CHUNK)], idx_v)
    pltpu.sync_copy(x_hbm.at[pl.ds(base, _CHUNK)], rows_v)
    pltpu.sync_copy(w_hbm.at[pl.ds(base, _CHUNK)], wrows_v)
    c1 = pltpu.async_copy(rows_v, xs_hbm.at[idx_v], sem)
    c2 = pltpu.async_copy(wrows_v, ws8_hbm.at[idx_v], sem2)
    c1.wait()
    c2.wait()

  return k(xf, dest, wrow)


def _combine(ys, dest):
  _NC, _NS, _CHUNK = _sc_geometry()
  mesh = plsc.VectorSubcoreMesh(core_axis_name="c", subcore_axis_name="s")

  @functools.partial(
      pl.kernel,
      mesh=mesh,
      out_type=jax.ShapeDtypeStruct((T, DIM), jnp.float32),
      scratch_types=[
          pltpu.VMEM((_CHUNK,), jnp.int32),
          pltpu.VMEM((_CHUNK, DIM), jnp.float32),
          pltpu.SemaphoreType.DMA,
      ],
  )
  def k(ys_hbm, dest_hbm, out_hbm, idx_v, rows_v, sem):
    wid = lax.axis_index("s") * _NC + lax.axis_index("c")
    base = wid * _CHUNK
    pltpu.sync_copy(dest_hbm.at[pl.ds(base, _CHUNK)], idx_v)
    pltpu.async_copy(ys_hbm.at[idx_v], rows_v, sem).wait()
    pltpu.sync_copy(rows_v, out_hbm.at[pl.ds(base, _CHUNK)])

  return k(ys, dest)


def kernel(x, router_w, fc1_w, fc1_b, gate_w, gate_b, fc2_w, fc2_b):
  Bq, Nq, C = x.shape
  xf = x.reshape(T, DIM)
  dest2d, wrow, bmap2d, aux, lb2d = _router(xf, router_w)
  dest = dest2d.reshape(T)
  bmap = bmap2d.reshape(NBLK)
  xs, ws8 = _dispatch(xf, dest, wrow)
  ys = _ffn(bmap, aux, xs, fc1_w, fc1_b.reshape(E, 1, DFF), gate_w,
            gate_b.reshape(E, 1, DFF), fc2_w, fc2_b.reshape(E, 1, DIM), ws8)
  out = _combine(ys, dest)
  return out.reshape(Bq, Nq, C), lb2d.reshape(())


# speedup vs baseline: 1.1671x; 1.1671x over previous
import functools

import jax
import jax.numpy as jnp
from jax import lax
from jax.experimental import pallas as pl
from jax.experimental.pallas import tpu as pltpu
from jax.experimental.pallas import tpu_sc as plsc

DIM = 768
E = 8
DFF = 1536
T = 2048
BLK = 128
NBLK = T // BLK + E
P = NBLK * BLK


def _cumsum_rows(x):
  n = x.shape[0]
  row = lax.broadcasted_iota(jnp.int32, x.shape, 0)
  s = 1
  while s < n:
    shifted = pltpu.roll(x, s, 0)
    x = x + jnp.where(row >= s, shifted, 0)
    s *= 2
  return x


def _cumsum_lanes(x):
  n = x.shape[1]
  col = lax.broadcasted_iota(jnp.int32, x.shape, 1)
  s = 1
  while s < n:
    shifted = pltpu.roll(x, s, 1)
    x = x + jnp.where(col >= s, shifted, 0)
    s *= 2
  return x


def _router_body(x_ref, rw_ref, dest_ref, w_ref, bmap_ref, aux_ref, lb_ref):
  xv = x_ref[...]
  rw = rw_ref[...]
  logits = lax.dot_general(xv, rw, (((1,), (1,)), ((), ())),
                           preferred_element_type=jnp.float32)
  m = jnp.max(logits, axis=1, keepdims=True)
  ex = jnp.exp(logits - m)
  probs = ex / jnp.sum(ex, axis=1, keepdims=True)
  pmax = jnp.max(probs, axis=1, keepdims=True)
  lane = lax.broadcasted_iota(jnp.int32, (T, E), 1)
  topi = jnp.min(jnp.where(probs == pmax, lane, E), axis=1, keepdims=True)
  w_ref[...] = jnp.broadcast_to(pmax / (pmax + 1e-6), (T, 128))

  onehot = (lane == topi).astype(jnp.int32)
  csum = _cumsum_rows(onehot)
  rank = jnp.sum((csum - onehot) * onehot, axis=1, keepdims=True)
  counts = jnp.sum(onehot, axis=0, keepdims=True)
  asz = ((counts + (BLK - 1)) // BLK) * BLK
  ends = _cumsum_lanes(asz)
  offs = ends - asz
  off_t = jnp.sum(offs * onehot, axis=1, keepdims=True)
  dest_ref[...] = off_t + rank

  bstart = lax.broadcasted_iota(jnp.int32, (NBLK, E), 0) * BLK
  bmap = jnp.sum((bstart >= jnp.broadcast_to(ends, (NBLK, E))).astype(jnp.int32),
                 axis=1, keepdims=True)
  bmap = jnp.minimum(bmap, E - 1)
  bmap_ref[...] = bmap

  rowb = lax.broadcasted_iota(jnp.int32, (NBLK, 1), 0)
  sw = ((bmap != pltpu.roll(bmap, 1, 0)) | (rowb == 0)).astype(jnp.int32)
  runidx = _cumsum_rows(sw) - 1
  buf = lax.rem(runidx, 2)
  lane_e = lax.broadcasted_iota(jnp.int32, (1, E), 1)
  has_pad = (ends[:, E - 1:E] < P).astype(jnp.int32)
  is_run_val = ((asz > 0) | ((lane_e == E - 1) & (has_pad > 0)))
  cand = jnp.where(is_run_val & (lane_e > bmap), lane_e, 99)
  nxt = jnp.min(cand, axis=1, keepdims=True)
  do_pref = ((sw == 1) & (nxt < 99)).astype(jnp.int32)
  aux_ref[...] = jnp.concatenate(
      [sw, buf, jnp.minimum(nxt, E - 1), do_pref], axis=1)

  psum = jnp.sum(probs, axis=0, keepdims=True)
  frac = counts.astype(jnp.float32) / (jnp.float32(T) + 1e-6)
  lb_ref[...] = jnp.sum(frac * psum, keepdims=True).reshape(1, 1) * E


def _router(xf, router_w):
  return pl.pallas_call(
      _router_body,
      out_shape=(
          jax.ShapeDtypeStruct((T, 1), jnp.int32),
          jax.ShapeDtypeStruct((T, 128), jnp.float32),
          jax.ShapeDtypeStruct((NBLK, 1), jnp.int32),
          jax.ShapeDtypeStruct((NBLK, 4), jnp.int32),
          jax.ShapeDtypeStruct((1, 1), jnp.float32),
      ),
  )(xf, router_w)


def _ffn_body(bmap_ref, aux_ref, xs_ref, f1w_hbm, f1b_ref, gw_hbm, gb_ref,
              f2w_hbm, f2b_ref, ws_ref, o_ref, w1_v, wg_v, w2_v, sems):
  i = pl.program_id(0)
  rs = aux_ref[i, 0]
  b = aux_ref[i, 1]
  ne = aux_ref[i, 2]
  dp = aux_ref[i, 3]
  cur_e = bmap_ref[i]

  def copies(e, slot):
    return (
        pltpu.make_async_copy(f1w_hbm.at[e], w1_v.at[slot], sems.at[slot, 0]),
        pltpu.make_async_copy(gw_hbm.at[e], wg_v.at[slot], sems.at[slot, 1]),
        pltpu.make_async_copy(f2w_hbm.at[e], w2_v.at[slot], sems.at[slot, 2]),
    )

  @pl.when(i == 0)
  def _():
    for c in copies(cur_e, b):
      c.start()

  @pl.when(rs == 1)
  def _():
    for c in copies(cur_e, b):
      c.wait()

    @pl.when(dp == 1)
    def _():
      for c in copies(ne, 1 - b):
        c.start()

  bf = jnp.bfloat16
  xb = xs_ref[...].astype(bf)
  nt = (((1,), (1,)), ((), ()))
  w1 = w1_v[pl.ds(b, 1)][0].astype(bf)
  wg = wg_v[pl.ds(b, 1)][0].astype(bf)
  w2 = w2_v[pl.ds(b, 1)][0].astype(bf)
  h = lax.dot_general(xb, w1, nt, preferred_element_type=jnp.float32)
  h = h + f1b_ref[0]
  g = lax.dot_general(xb, wg, nt, preferred_element_type=jnp.float32)
  g = g + gb_ref[0]
  a = g * jax.nn.sigmoid(g) * h
  o = lax.dot_general(a.astype(bf), w2, nt, preferred_element_type=jnp.float32)
  o_ref[...] = (o + f2b_ref[0]) * ws_ref[...][:, 0:1]


def _ffn(bmap, aux, xs, fc1_w, fc1_b, gate_w, gate_b, fc2_w, fc2_b, ws8):
  grid_spec = pltpu.PrefetchScalarGridSpec(
      num_scalar_prefetch=2,
      grid=(NBLK,),
      in_specs=[
          pl.BlockSpec((BLK, DIM), lambda i, bm, ax: (i, 0)),
          pl.BlockSpec(memory_space=pl.MemorySpace.ANY),
          pl.BlockSpec((1, 1, DFF), lambda i, bm, ax: (bm[i], 0, 0)),
          pl.BlockSpec(memory_space=pl.MemorySpace.ANY),
          pl.BlockSpec((1, 1, DFF), lambda i, bm, ax: (bm[i], 0, 0)),
          pl.BlockSpec(memory_space=pl.MemorySpace.ANY),
          pl.BlockSpec((1, 1, DIM), lambda i, bm, ax: (bm[i], 0, 0)),
          pl.BlockSpec((BLK, 128), lambda i, bm, ax: (i, 0)),
      ],
      out_specs=pl.BlockSpec((BLK, DIM), lambda i, bm, ax: (i, 0)),
      scratch_shapes=[
          pltpu.VMEM((2, DFF, DIM), jnp.float32),
          pltpu.VMEM((2, DFF, DIM), jnp.float32),
          pltpu.VMEM((2, DIM, DFF), jnp.float32),
          pltpu.SemaphoreType.DMA((2, 3)),
      ],
  )
  return pl.pallas_call(
      _ffn_body,
      grid_spec=grid_spec,
      out_shape=jax.ShapeDtypeStruct((P, DIM), jnp.float32),
  )(bmap, aux, xs, fc1_w, fc1_b, gate_w, gate_b, fc2_w, fc2_b, ws8)


@functools.cache
def _sc_geometry():
  info = plsc.get_sparse_core_info()
  nc, ns = info.num_cores, info.num_subcores
  return nc, ns, T // (nc * ns)


def _dispatch(xf, dest, wrow):
  _NC, _NS, _CHUNK = _sc_geometry()
  mesh = plsc.VectorSubcoreMesh(core_axis_name="c", subcore_axis_name="s")

  @functools.partial(
      pl.kernel,
      mesh=mesh,
      out_type=(
          jax.ShapeDtypeStruct((P, DIM), jnp.float32),
          jax.ShapeDtypeStruct((P, 128), jnp.float32),
      ),
      scratch_types=[
          pltpu.VMEM((_CHUNK,), jnp.int32),
          pltpu.VMEM((_CHUNK, DIM), jnp.float32),
          pltpu.VMEM((_CHUNK, 128), jnp.float32),
          pltpu.SemaphoreType.DMA,
          pltpu.SemaphoreType.DMA,
      ],
  )
  def k(x_hbm, dest_hbm, w_hbm, xs_hbm, ws8_hbm, idx_v, rows_v, wrows_v, sem,
        sem2):
    wid = lax.axis_index("s") * _NC + lax.axis_index("c")
    base = wid * _CHUNK
    pltpu.sync_copy(dest_hbm.at[pl.ds(base, _CHUNK)], idx_v)
    pltpu.sync_copy(x_hbm.at[pl.ds(base, _CHUNK)], rows_v)
    pltpu.sync_copy(w_hbm.at[pl.ds(base, _CHUNK)], wrows_v)
    c1 = pltpu.async_copy(rows_v, xs_hbm.at[idx_v], sem)
    c2 = pltpu.async_copy(wrows_v, ws8_hbm.at[idx_v], sem2)
    c1.wait()
    c2.wait()

  return k(xf, dest, wrow)


def _combine(ys, dest):
  _NC, _NS, _CHUNK = _sc_geometry()
  mesh = plsc.VectorSubcoreMesh(core_axis_name="c", subcore_axis_name="s")

  @functools.partial(
      pl.kernel,
      mesh=mesh,
      out_type=jax.ShapeDtypeStruct((T, DIM), jnp.float32),
      scratch_types=[
          pltpu.VMEM((_CHUNK,), jnp.int32),
          pltpu.VMEM((_CHUNK, DIM), jnp.float32),
          pltpu.SemaphoreType.DMA,
      ],
  )
  def k(ys_hbm, dest_hbm, out_hbm, idx_v, rows_v, sem):
    wid = lax.axis_index("s") * _NC + lax.axis_index("c")
    base = wid * _CHUNK
    pltpu.sync_copy(dest_hbm.at[pl.ds(base, _CHUNK)], idx_v)
    pltpu.async_copy(ys_hbm.at[idx_v], rows_v, sem).wait()
    pltpu.sync_copy(rows_v, out_hbm.at[pl.ds(base, _CHUNK)])

  return k(ys, dest)


def kernel(x, router_w, fc1_w, fc1_b, gate_w, gate_b, fc2_w, fc2_b):
  Bq, Nq, C = x.shape
  xf = x.reshape(T, DIM)
  dest2d, wrow, bmap2d, aux, lb2d = _router(xf, router_w)
  dest = dest2d.reshape(T)
  bmap = bmap2d.reshape(NBLK)
  xs, ws8 = _dispatch(xf, dest, wrow)
  ys = _ffn(bmap, aux, xs, fc1_w, fc1_b.reshape(E, 1, DFF), gate_w,
            gate_b.reshape(E, 1, DFF), fc2_w, fc2_b.reshape(E, 1, DIM), ws8)
  out = _combine(ys, dest)
  return out.reshape(Bq, Nq, C), lb2d.reshape(())


# speedup vs baseline: 1.2585x; 1.0783x over previous
import functools

import jax
import jax.numpy as jnp
from jax import lax
from jax.experimental import pallas as pl
from jax.experimental.pallas import tpu as pltpu
from jax.experimental.pallas import tpu_sc as plsc

DIM = 768
E = 8
DFF = 1536
T = 2048
BLK = 128
NBLK = T // BLK + E
P = NBLK * BLK


def _cumsum_rows(x):
  n = x.shape[0]
  row = lax.broadcasted_iota(jnp.int32, x.shape, 0)
  s = 1
  while s < n:
    shifted = pltpu.roll(x, s, 0)
    x = x + jnp.where(row >= s, shifted, 0)
    s *= 2
  return x


def _cumsum_lanes(x):
  n = x.shape[1]
  col = lax.broadcasted_iota(jnp.int32, x.shape, 1)
  s = 1
  while s < n:
    shifted = pltpu.roll(x, s, 1)
    x = x + jnp.where(col >= s, shifted, 0)
    s *= 2
  return x


def _router_body(x_ref, rw_ref, dest_ref, w_ref, bmap_ref, aux_ref, lb_ref):
  xv = x_ref[...]
  rw = rw_ref[...]
  logits = lax.dot_general(xv, rw, (((1,), (1,)), ((), ())),
                           preferred_element_type=jnp.float32)
  m = jnp.max(logits, axis=1, keepdims=True)
  ex = jnp.exp(logits - m)
  probs = ex / jnp.sum(ex, axis=1, keepdims=True)
  pmax = jnp.max(probs, axis=1, keepdims=True)
  lane = lax.broadcasted_iota(jnp.int32, (T, E), 1)
  topi = jnp.min(jnp.where(probs == pmax, lane, E), axis=1, keepdims=True)
  w_ref[...] = jnp.broadcast_to(pmax / (pmax + 1e-6), (T, 128))

  onehot = (lane == topi).astype(jnp.int32)
  csum = _cumsum_rows(onehot)
  rank = jnp.sum((csum - onehot) * onehot, axis=1, keepdims=True)
  counts = jnp.sum(onehot, axis=0, keepdims=True)
  asz = ((counts + (BLK - 1)) // BLK) * BLK
  ends = _cumsum_lanes(asz)
  offs = ends - asz
  off_t = jnp.sum(offs * onehot, axis=1, keepdims=True)
  dest_ref[...] = off_t + rank

  bstart = lax.broadcasted_iota(jnp.int32, (NBLK, E), 0) * BLK
  bmap = jnp.sum((bstart >= jnp.broadcast_to(ends, (NBLK, E))).astype(jnp.int32),
                 axis=1, keepdims=True)
  bmap = jnp.minimum(bmap, E - 1)
  bmap_ref[...] = bmap

  rowb = lax.broadcasted_iota(jnp.int32, (NBLK, 1), 0)
  sw = ((bmap != pltpu.roll(bmap, 1, 0)) | (rowb == 0)).astype(jnp.int32)
  runidx = _cumsum_rows(sw) - 1
  buf = lax.rem(runidx, 2)
  lane_e = lax.broadcasted_iota(jnp.int32, (1, E), 1)
  is_run_val = (asz > 0)
  cand = jnp.where(is_run_val & (lane_e > bmap), lane_e, 99)
  nxt = jnp.min(cand, axis=1, keepdims=True)
  do_pref = ((sw == 1) & (nxt < 99)).astype(jnp.int32)
  valid = (rowb * BLK < ends[:, E - 1:E]).astype(jnp.int32)
  aux_ref[...] = jnp.concatenate(
      [sw, buf, jnp.minimum(nxt, E - 1), do_pref, valid], axis=1)

  psum = jnp.sum(probs, axis=0, keepdims=True)
  frac = counts.astype(jnp.float32) / (jnp.float32(T) + 1e-6)
  lb_ref[...] = jnp.sum(frac * psum, keepdims=True).reshape(1, 1) * E


def _router(xf, router_w):
  return pl.pallas_call(
      _router_body,
      out_shape=(
          jax.ShapeDtypeStruct((T, 1), jnp.int32),
          jax.ShapeDtypeStruct((T, 128), jnp.float32),
          jax.ShapeDtypeStruct((NBLK, 1), jnp.int32),
          jax.ShapeDtypeStruct((NBLK, 5), jnp.int32),
          jax.ShapeDtypeStruct((1, 1), jnp.float32),
      ),
  )(xf, router_w)


def _ffn_body(bmap_ref, aux_ref, xs_ref, f1w_hbm, f1b_ref, gw_hbm, gb_ref,
              f2w_hbm, f2b_ref, ws_ref, o_ref, w1_v, wg_v, w2_v, sems):
  i = pl.program_id(0)
  rs = aux_ref[i, 0]
  b = aux_ref[i, 1]
  ne = aux_ref[i, 2]
  dp = aux_ref[i, 3]
  valid = aux_ref[i, 4]
  cur_e = bmap_ref[i]

  def copies(e, slot):
    return (
        pltpu.make_async_copy(f1w_hbm.at[e], w1_v.at[slot], sems.at[slot, 0]),
        pltpu.make_async_copy(gw_hbm.at[e], wg_v.at[slot], sems.at[slot, 1]),
        pltpu.make_async_copy(f2w_hbm.at[e], w2_v.at[slot], sems.at[slot, 2]),
    )

  @pl.when(i == 0)
  def _():
    for c in copies(cur_e, b):
      c.start()

  @pl.when((rs == 1) & (valid == 1))
  def _():
    for c in copies(cur_e, b):
      c.wait()

    @pl.when(dp == 1)
    def _():
      for c in copies(ne, 1 - b):
        c.start()

  @pl.when(valid == 1)
  def _():
    bf = jnp.bfloat16
    xb = xs_ref[...].astype(bf)
    nt = (((1,), (1,)), ((), ()))
    w1 = w1_v[pl.ds(b, 1)][0].astype(bf)
    wg = wg_v[pl.ds(b, 1)][0].astype(bf)
    w2 = w2_v[pl.ds(b, 1)][0].astype(bf)
    h = lax.dot_general(xb, w1, nt, preferred_element_type=jnp.float32)
    h = h + f1b_ref[0]
    g = lax.dot_general(xb, wg, nt, preferred_element_type=jnp.float32)
    g = g + gb_ref[0]
    a = g * jax.nn.sigmoid(g) * h
    o = lax.dot_general(a.astype(bf), w2, nt,
                        preferred_element_type=jnp.float32)
    o_ref[...] = (o + f2b_ref[0]) * ws_ref[...][:, 0:1]


def _ffn(bmap, aux, xs, fc1_w, fc1_b, gate_w, gate_b, fc2_w, fc2_b, ws8):
  grid_spec = pltpu.PrefetchScalarGridSpec(
      num_scalar_prefetch=2,
      grid=(NBLK,),
      in_specs=[
          pl.BlockSpec((BLK, DIM), lambda i, bm, ax: (i, 0)),
          pl.BlockSpec(memory_space=pl.MemorySpace.ANY),
          pl.BlockSpec((1, 1, DFF), lambda i, bm, ax: (bm[i], 0, 0)),
          pl.BlockSpec(memory_space=pl.MemorySpace.ANY),
          pl.BlockSpec((1, 1, DFF), lambda i, bm, ax: (bm[i], 0, 0)),
          pl.BlockSpec(memory_space=pl.MemorySpace.ANY),
          pl.BlockSpec((1, 1, DIM), lambda i, bm, ax: (bm[i], 0, 0)),
          pl.BlockSpec((BLK, 128), lambda i, bm, ax: (i, 0)),
      ],
      out_specs=pl.BlockSpec((BLK, DIM), lambda i, bm, ax: (i, 0)),
      scratch_shapes=[
          pltpu.VMEM((2, DFF, DIM), jnp.float32),
          pltpu.VMEM((2, DFF, DIM), jnp.float32),
          pltpu.VMEM((2, DIM, DFF), jnp.float32),
          pltpu.SemaphoreType.DMA((2, 3)),
      ],
  )
  return pl.pallas_call(
      _ffn_body,
      grid_spec=grid_spec,
      out_shape=jax.ShapeDtypeStruct((P, DIM), jnp.float32),
  )(bmap, aux, xs, fc1_w, fc1_b, gate_w, gate_b, fc2_w, fc2_b, ws8)


@functools.cache
def _sc_geometry():
  info = plsc.get_sparse_core_info()
  nc, ns = info.num_cores, info.num_subcores
  return nc, ns, T // (nc * ns)


def _dispatch(xf, dest, wrow):
  _NC, _NS, _CHUNK = _sc_geometry()
  mesh = plsc.VectorSubcoreMesh(core_axis_name="c", subcore_axis_name="s")

  @functools.partial(
      pl.kernel,
      mesh=mesh,
      out_type=(
          jax.ShapeDtypeStruct((P, DIM), jnp.float32),
          jax.ShapeDtypeStruct((P, 128), jnp.float32),
      ),
      scratch_types=[
          pltpu.VMEM((_CHUNK,), jnp.int32),
          pltpu.VMEM((_CHUNK, DIM), jnp.float32),
          pltpu.VMEM((_CHUNK, 128), jnp.float32),
          pltpu.SemaphoreType.DMA,
          pltpu.SemaphoreType.DMA,
      ],
  )
  def k(x_hbm, dest_hbm, w_hbm, xs_hbm, ws8_hbm, idx_v, rows_v, wrows_v, sem,
        sem2):
    wid = lax.axis_index("s") * _NC + lax.axis_index("c")
    base = wid * _CHUNK
    pltpu.sync_copy(dest_hbm.at[pl.ds(base, _CHUNK)], idx_v)
    pltpu.sync_copy(x_hbm.at[pl.ds(base, _CHUNK)], rows_v)
    pltpu.sync_copy(w_hbm.at[pl.ds(base, _CHUNK)], wrows_v)
    c1 = pltpu.async_copy(rows_v, xs_hbm.at[idx_v], sem)
    c2 = pltpu.async_copy(wrows_v, ws8_hbm.at[idx_v], sem2)
    c1.wait()
    c2.wait()

  return k(xf, dest, wrow)


def _combine(ys, dest):
  _NC, _NS, _CHUNK = _sc_geometry()
  mesh = plsc.VectorSubcoreMesh(core_axis_name="c", subcore_axis_name="s")

  @functools.partial(
      pl.kernel,
      mesh=mesh,
      out_type=jax.ShapeDtypeStruct((T, DIM), jnp.float32),
      scratch_types=[
          pltpu.VMEM((_CHUNK,), jnp.int32),
          pltpu.VMEM((_CHUNK, DIM), jnp.float32),
          pltpu.SemaphoreType.DMA,
      ],
  )
  def k(ys_hbm, dest_hbm, out_hbm, idx_v, rows_v, sem):
    wid = lax.axis_index("s") * _NC + lax.axis_index("c")
    base = wid * _CHUNK
    pltpu.sync_copy(dest_hbm.at[pl.ds(base, _CHUNK)], idx_v)
    pltpu.async_copy(ys_hbm.at[idx_v], rows_v, sem).wait()
    pltpu.sync_copy(rows_v, out_hbm.at[pl.ds(base, _CHUNK)])

  return k(ys, dest)


def kernel(x, router_w, fc1_w, fc1_b, gate_w, gate_b, fc2_w, fc2_b):
  Bq, Nq, C = x.shape
  xf = x.reshape(T, DIM)
  dest2d, wrow, bmap2d, aux, lb2d = _router(xf, router_w)
  dest = dest2d.reshape(T)
  bmap = bmap2d.reshape(NBLK)
  xs, ws8 = _dispatch(xf, dest, wrow)
  ys = _ffn(bmap, aux, xs, fc1_w, fc1_b.reshape(E, 1, DFF), gate_w,
            gate_b.reshape(E, 1, DFF), fc2_w, fc2_b.reshape(E, 1, DIM), ws8)
  out = _combine(ys, dest)
  return out.reshape(Bq, Nq, C), lb2d.reshape(())
